# baseline (speedup 1.0000x reference)
import jax
import jax.numpy as jnp
from jax import lax
from jax.experimental import pallas as pl


def _fw_kernel(f_ref, w_ref, o_ref):
    o_ref[...] = lax.dot_general(
        f_ref[...], w_ref[...], (((1,), (0,)), ((), ())),
        preferred_element_type=jnp.float32)


def _zzt_kernel(zi_ref, zj_ref, o_ref):
    o_ref[...] = lax.dot_general(
        zi_ref[...], zj_ref[...], (((1,), (1,)), ((), ())),
        preferred_element_type=jnp.float32)


def kernel(features, edge_index, node_ids, W, bias, alpha):
    n, f = features.shape
    h_dim = W.shape[1]
    e = edge_index.shape[1]
    gene_num = alpha.shape[0] - 2
    src = edge_index[0]
    dst = edge_index[1]

    out_deg = jnp.clip(jnp.zeros((n,), jnp.float32).at[src].add(1.0), 1.0, None)

    bm = 1024
    p = pl.pallas_call(
        _fw_kernel,
        grid=(pl.cdiv(n, bm),),
        in_specs=[pl.BlockSpec((bm, f), lambda i: (i, 0)),
                  pl.BlockSpec((f, h_dim), lambda i: (0, 0))],
        out_specs=pl.BlockSpec((bm, h_dim), lambda i: (i, 0)),
        out_shape=jax.ShapeDtypeStruct((n, h_dim), jnp.float32),
    )(features, W)
    h = p * (out_deg ** -0.5)[:, None]

    src_id = node_ids[src]
    dst_id = node_ids[dst]
    idx = jnp.full((e,), gene_num + 1, dtype=jnp.int32)
    idx = jnp.where((src_id >= 0) & (dst_id < 0), src_id, idx)
    idx = jnp.where((dst_id >= 0) & (src_id < 0), dst_id, idx)
    idx = jnp.where((dst_id >= 0) & (src_id >= 0), jnp.int32(gene_num), idx)

    m = h[src] * alpha[idx]
    rst = jnp.zeros((n, h_dim), jnp.float32).at[dst].add(m)
    in_deg = jnp.clip(jnp.zeros((n,), jnp.float32).at[dst].add(1.0), 1.0, None)
    z = rst * (in_deg ** -0.5)[:, None] + bias

    bz = 1024
    adj = pl.pallas_call(
        _zzt_kernel,
        grid=(pl.cdiv(n, bz), pl.cdiv(n, bz)),
        in_specs=[pl.BlockSpec((bz, h_dim), lambda i, j: (i, 0)),
                  pl.BlockSpec((bz, h_dim), lambda i, j: (j, 0))],
        out_specs=pl.BlockSpec((bz, bz), lambda i, j: (i, j)),
        out_shape=jax.ShapeDtypeStruct((n, n), jnp.float32),
    )(z, z)
    return (adj, z)


# speedup vs baseline: 5.1555x; 5.1555x over previous
import functools

import jax
import jax.numpy as jnp
from jax import lax
from jax.experimental import pallas as pl
from jax.experimental.pallas import tpu as pltpu
from jax.experimental.pallas import tpu_sc as plsc

_N = 10000
_E = 160000
_NSUB = 16
_ESUB = 10112
_NCHUNK = 79
_CW = 128
_QW = 64
_ACC_ROWS = 20008
_SLAB = 1248
_SLAB_EXTRA = 2 * _N - _NSUB * _SLAB


def _fw_table_kernel(f_ref, w_ref, so_ref, av_ref, o_ref):
    p = lax.dot_general(f_ref[...], w_ref[...], (((1,), (0,)), ((), ())),
                        preferred_element_type=jnp.float32)
    hb = p * so_ref[...]
    h1 = hb * av_ref[...]
    bm = hb.shape[0]
    hbq = hb.reshape(bm, 4, _QW).transpose(1, 0, 2)
    h1q = h1.reshape(bm, 4, _QW).transpose(1, 0, 2)
    o_ref[...] = jnp.stack([hbq, h1q], axis=1)


def _combine_kernel(ap_ref, aq_ref, wp_ref, wq_ref, si_ref, b_ref, z_ref):
    bm = ap_ref.shape[1]
    ap = ap_ref[...].transpose(1, 0, 2).reshape(bm, 4 * _QW)
    aq = aq_ref[...].transpose(1, 0, 2).reshape(bm, 4 * _QW)
    z_ref[...] = si_ref[...] * (wp_ref[...] * ap + wq_ref[...] * aq) + b_ref[...]


def _zzt_kernel(zi_ref, zj_ref, o_ref):
    o_ref[...] = lax.dot_general(
        zi_ref[...], zj_ref[...], (((1,), (1,)), ((), ())),
        preferred_element_type=jnp.float32)


def _edge_agg(h2, srcp, dstp, node_ids, zeros):
    mesh = plsc.VectorSubcoreMesh(core_axis_name="c", subcore_axis_name="s")

    @functools.partial(
        pl.kernel,
        mesh=mesh,
        compiler_params=pltpu.CompilerParams(needs_layout_passes=False,
                                             use_tc_tiling_on_sc=False),
        out_type=jax.ShapeDtypeStruct((4, 2 * _N, _QW), jnp.float32),
        scratch_types=[
            pltpu.VMEM((_N,), jnp.int32),
            pltpu.VMEM((_NCHUNK, _CW), jnp.int32),
            pltpu.VMEM((_NCHUNK, _CW), jnp.int32),
            pltpu.VMEM((_CW, _QW), jnp.float32),
            pltpu.VMEM((_CW, _QW), jnp.float32),
            pltpu.VMEM_SHARED((_ACC_ROWS, _QW), jnp.float32),
            pltpu.SemaphoreType.DMA,
            pltpu.SemaphoreType.DMA,
        ],
    )
    def agg(h2_hbm, src_hbm, dst_hbm, nid_hbm, zero_hbm, acc_hbm,
            node_v, gidx_v, aidx_v, buf0, buf1, acc_sh, sem0, sem1):
        cid = lax.axis_index("c")
        sid = lax.axis_index("s")
        slab = sid * _SLAB

        pltpu.sync_copy(nid_hbm, node_v)
        pltpu.sync_copy(src_hbm.at[sid], gidx_v)
        pltpu.sync_copy(dst_hbm.at[sid], aidx_v)

        base = cid * 2 * (2 * _N)

        def idx_body(j, carry):
            for k in range(_CW // 16):
                ds = pl.ds(k * 16, 16)
                s = gidx_v[j, ds]
                d = aidx_v[j, ds]
                s_id = plsc.load_gather(node_v, [s])
                d_id = plsc.load_gather(node_v, [jnp.maximum(d, 0)])
                t = (s_id >= 0) & (d_id < 0)
                gidx_v[j, ds] = base + s + jnp.where(t, _N, 0)
                aidx_v[j, ds] = jnp.where(d < 0, 2 * _N,
                                          d + jnp.where(s_id < 0, _N, 0))
            return carry

        def bump_body(j, carry):
            for k in range(_CW // 16):
                ds = pl.ds(k * 16, 16)
                gidx_v[j, ds] = gidx_v[j, ds] + 2 * _N
            return carry

        for q in range(2):
            qg = cid * 2 + q
            lax.fori_loop(0, _NCHUNK, idx_body if q == 0 else bump_body, 0)

            pltpu.sync_copy(zero_hbm, acc_sh.at[pl.ds(slab, _SLAB)])

            @pl.when(sid == _NSUB - 1)
            def _():
                pltpu.sync_copy(zero_hbm.at[pl.ds(0, _SLAB_EXTRA)],
                                acc_sh.at[pl.ds(_NSUB * _SLAB, _SLAB_EXTRA)])

            plsc.subcore_barrier()

            bufs = (buf0, buf1)
            sems = (sem0, sem1)
            prev = pltpu.async_copy(h2_hbm.at[gidx_v.at[0]], bufs[0], sems[0])
            for j in range(1, _NCHUNK):
                cur = pltpu.async_copy(h2_hbm.at[gidx_v.at[j]],
                                       bufs[j % 2], sems[j % 2])
                prev.wait()
                pltpu.sync_copy(bufs[(j - 1) % 2],
                                acc_sh.at[aidx_v.at[j - 1]], add=True)
                prev = cur
            prev.wait()
            pltpu.sync_copy(bufs[(_NCHUNK - 1) % 2],
                            acc_sh.at[aidx_v.at[_NCHUNK - 1]], add=True)

            plsc.subcore_barrier()
            pltpu.sync_copy(acc_sh.at[pl.ds(slab, _SLAB)],
                            acc_hbm.at[qg, pl.ds(slab, _SLAB)])

            @pl.when(sid == _NSUB - 1)
            def _():
                pltpu.sync_copy(
                    acc_sh.at[pl.ds(_NSUB * _SLAB, _SLAB_EXTRA)],
                    acc_hbm.at[qg, pl.ds(_NSUB * _SLAB, _SLAB_EXTRA)])

    return agg(h2, srcp, dstp, node_ids, zeros)


def kernel(features, edge_index, node_ids, W, bias, alpha):
    n, f = features.shape
    h_dim = W.shape[1]
    gene_num = alpha.shape[0] - 2
    src = edge_index[0]
    dst = edge_index[1]

    out_deg = jnp.clip(jnp.zeros((n,), jnp.float32).at[src].add(1.0), 1.0, None)
    in_deg = jnp.clip(jnp.zeros((n,), jnp.float32).at[dst].add(1.0), 1.0, None)
    so = (out_deg ** -0.5)[:, None]
    si = (in_deg ** -0.5)[:, None]

    is_gene = node_ids >= 0
    av = jnp.where(is_gene, alpha[jnp.maximum(node_ids, 0), 0], 1.0)[:, None]
    c3 = alpha[gene_num, 0]
    c4 = alpha[gene_num + 1, 0]
    wp = jnp.where(is_gene, c3, 1.0)[:, None]
    wq = jnp.where(is_gene, av[:, 0], c4)[:, None]

    bm = 1000
    h2 = pl.pallas_call(
        _fw_table_kernel,
        grid=(n // bm,),
        in_specs=[pl.BlockSpec((bm, f), lambda i: (i, 0)),
                  pl.BlockSpec((f, h_dim), lambda i: (0, 0)),
                  pl.BlockSpec((bm, 1), lambda i: (i, 0)),
                  pl.BlockSpec((bm, 1), lambda i: (i, 0))],
        out_specs=pl.BlockSpec((4, 2, bm, _QW), lambda i: (0, 0, i, 0)),
        out_shape=jax.ShapeDtypeStruct((4, 2, n, _QW), jnp.float32),
    )(features, W, so, av)
    h2 = h2.reshape(4 * 2 * n, _QW)

    srcp = jnp.pad(src.reshape(_NSUB, _E // _NSUB),
                   ((0, 0), (0, _ESUB - _E // _NSUB))).reshape(
                       _NSUB, _NCHUNK, _CW)
    dstp = jnp.pad(dst.reshape(_NSUB, _E // _NSUB),
                   ((0, 0), (0, _ESUB - _E // _NSUB)),
                   constant_values=-1).reshape(_NSUB, _NCHUNK, _CW)
    zeros = jnp.zeros((_SLAB, _QW), jnp.float32)

    acc = _edge_agg(h2, srcp, dstp, node_ids, zeros)

    z = pl.pallas_call(
        _combine_kernel,
        grid=(n // bm,),
        in_specs=[pl.BlockSpec((4, bm, _QW), lambda i: (0, i, 0)),
                  pl.BlockSpec((4, bm, _QW), lambda i: (0, i + _N // 1000, 0)),
                  pl.BlockSpec((bm, 1), lambda i: (i, 0)),
                  pl.BlockSpec((bm, 1), lambda i: (i, 0)),
                  pl.BlockSpec((bm, 1), lambda i: (i, 0)),
                  pl.BlockSpec((1, h_dim), lambda i: (0, 0))],
        out_specs=pl.BlockSpec((bm, h_dim), lambda i: (i, 0)),
        out_shape=jax.ShapeDtypeStruct((n, h_dim), jnp.float32),
    )(acc, acc, wp, wq, si, bias[None, :])

    bz = 1024
    adj = pl.pallas_call(
        _zzt_kernel,
        grid=(pl.cdiv(n, bz), pl.cdiv(n, bz)),
        in_specs=[pl.BlockSpec((bz, h_dim), lambda i, j: (i, 0)),
                  pl.BlockSpec((bz, h_dim), lambda i, j: (j, 0))],
        out_specs=pl.BlockSpec((bz, bz), lambda i, j: (i, j)),
        out_shape=jax.ShapeDtypeStruct((n, n), jnp.float32),
    )(z, z)
    return (adj, z)


# speedup vs baseline: 5.2748x; 1.0231x over previous
import functools

import jax
import jax.numpy as jnp
from jax import lax
from jax.experimental import pallas as pl
from jax.experimental.pallas import tpu as pltpu
from jax.experimental.pallas import tpu_sc as plsc

_N = 10000
_E = 160000
_NSUB = 16
_ESUB = 10112
_NCHUNK = 79
_CW = 128
_QW = 64
_ACC_ROWS = 20008
_NBUF = 3
_BITS_W = 320
_SLAB = 1248
_SLAB_EXTRA = 2 * _N - _NSUB * _SLAB


def _fw_table_kernel(f_ref, w_ref, so_ref, av_ref, o_ref):
    p = lax.dot_general(f_ref[...], w_ref[...], (((1,), (0,)), ((), ())),
                        preferred_element_type=jnp.float32)
    hb = p * so_ref[...]
    h1 = hb * av_ref[...]
    bm = hb.shape[0]
    hbq = hb.reshape(bm, 4, _QW).transpose(1, 0, 2)
    h1q = h1.reshape(bm, 4, _QW).transpose(1, 0, 2)
    o_ref[...] = jnp.stack([hbq, h1q], axis=1)


def _combine_kernel(ap_ref, aq_ref, wp_ref, wq_ref, si_ref, b_ref, z_ref):
    bm = ap_ref.shape[1]
    ap = ap_ref[...].transpose(1, 0, 2).reshape(bm, 4 * _QW)
    aq = aq_ref[...].transpose(1, 0, 2).reshape(bm, 4 * _QW)
    z_ref[...] = si_ref[...] * (wp_ref[...] * ap + wq_ref[...] * aq) + b_ref[...]


def _zzt_kernel(zi_ref, zj_ref, o_ref):
    o_ref[...] = lax.dot_general(
        zi_ref[...], zj_ref[...], (((1,), (1,)), ((), ())),
        preferred_element_type=jnp.float32)


def _edge_agg(h2, srcp, dstp, bits, zeros):
    mesh = plsc.VectorSubcoreMesh(core_axis_name="c", subcore_axis_name="s")

    @functools.partial(
        pl.kernel,
        mesh=mesh,
        compiler_params=pltpu.CompilerParams(needs_layout_passes=False,
                                             use_tc_tiling_on_sc=False),
        out_type=jax.ShapeDtypeStruct((4, 2 * _N, _QW), jnp.float32),
        scratch_types=[
            pltpu.VMEM((_BITS_W,), jnp.int32),
            pltpu.VMEM((_NCHUNK, _CW), jnp.int32),
            pltpu.VMEM((_NCHUNK, _CW), jnp.int32),
            pltpu.VMEM((_NBUF, _CW, _QW), jnp.float32),
            pltpu.VMEM_SHARED((_ACC_ROWS, _QW), jnp.float32),
            [pltpu.SemaphoreType.DMA] * _NBUF,
            [pltpu.SemaphoreType.DMA] * _NBUF,
        ],
    )
    def agg(h2_hbm, src_hbm, dst_hbm, bits_hbm, zero_hbm, acc_hbm,
            bits_v, gidx_v, aidx_v, bufs_v, acc_sh, gsems, ssems):
        cid = lax.axis_index("c")
        sid = lax.axis_index("s")
        slab = sid * _SLAB

        pltpu.sync_copy(bits_hbm, bits_v)
        pltpu.sync_copy(src_hbm.at[sid], gidx_v)
        pltpu.sync_copy(dst_hbm.at[sid], aidx_v)

        base = cid * 2 * (2 * _N)

        def idx_body(j, carry):
            for k in range(_CW // 16):
                ds = pl.ds(k * 16, 16)
                s = gidx_v[j, ds]
                d = aidx_v[j, ds]
                dc = jnp.maximum(d, 0)
                sw = plsc.load_gather(bits_v, [lax.shift_right_logical(s, 5)])
                dw = plsc.load_gather(bits_v, [lax.shift_right_logical(dc, 5)])
                s_gene = lax.shift_right_logical(sw, s & 31) & 1
                d_gene = lax.shift_right_logical(dw, dc & 31) & 1
                t = (s_gene == 1) & (d_gene == 0)
                gidx_v[j, ds] = base + s + jnp.where(t, _N, 0)
                aidx_v[j, ds] = jnp.where(d < 0, 2 * _N,
                                          d + jnp.where(s_gene == 0, _N, 0))
            return carry

        def bump_body(j, carry):
            for k in range(_CW // 16):
                ds = pl.ds(k * 16, 16)
                gidx_v[j, ds] = gidx_v[j, ds] + 2 * _N
            return carry

        for q in range(2):
            qg = cid * 2 + q
            lax.fori_loop(0, _NCHUNK, idx_body if q == 0 else bump_body, 0)

            pltpu.sync_copy(zero_hbm, acc_sh.at[pl.ds(slab, _SLAB)])

            @pl.when(sid == _NSUB - 1)
            def _():
                pltpu.sync_copy(zero_hbm.at[pl.ds(0, _SLAB_EXTRA)],
                                acc_sh.at[pl.ds(_NSUB * _SLAB, _SLAB_EXTRA)])

            plsc.subcore_barrier()

            ghand = [None] * _NCHUNK
            shand = [None] * _NCHUNK

            def start_gather(j):
                return pltpu.async_copy(h2_hbm.at[gidx_v.at[j]],
                                        bufs_v.at[j % _NBUF],
                                        gsems[j % _NBUF])

            def start_scatter(j):
                return pltpu.async_copy(bufs_v.at[j % _NBUF],
                                        acc_sh.at[aidx_v.at[j]],
                                        ssems[j % _NBUF], add=True)

            for j in range(_NCHUNK):
                if j >= _NBUF:
                    shand[j - _NBUF].wait()
                ghand[j] = start_gather(j)
                if j >= 2:
                    ghand[j - 2].wait()
                    shand[j - 2] = start_scatter(j - 2)
            for j in range(_NCHUNK - 2, _NCHUNK):
                ghand[j].wait()
                shand[j] = start_scatter(j)
            for j in range(_NCHUNK - _NBUF, _NCHUNK):
                shand[j].wait()

            plsc.subcore_barrier()
            pltpu.sync_copy(acc_sh.at[pl.ds(slab, _SLAB)],
                            acc_hbm.at[qg, pl.ds(slab, _SLAB)])

            @pl.when(sid == _NSUB - 1)
            def _():
                pltpu.sync_copy(
                    acc_sh.at[pl.ds(_NSUB * _SLAB, _SLAB_EXTRA)],
                    acc_hbm.at[qg, pl.ds(_NSUB * _SLAB, _SLAB_EXTRA)])

    return agg(h2, srcp, dstp, bits, zeros)


def kernel(features, edge_index, node_ids, W, bias, alpha):
    n, f = features.shape
    h_dim = W.shape[1]
    gene_num = alpha.shape[0] - 2
    src = edge_index[0]
    dst = edge_index[1]

    out_deg = jnp.clip(jnp.zeros((n,), jnp.float32).at[src].add(1.0), 1.0, None)
    in_deg = jnp.clip(jnp.zeros((n,), jnp.float32).at[dst].add(1.0), 1.0, None)
    so = (out_deg ** -0.5)[:, None]
    si = (in_deg ** -0.5)[:, None]

    is_gene = node_ids >= 0
    av = jnp.where(is_gene, alpha[jnp.maximum(node_ids, 0), 0], 1.0)[:, None]
    c3 = alpha[gene_num, 0]
    c4 = alpha[gene_num + 1, 0]
    wp = jnp.where(is_gene, c3, 1.0)[:, None]
    wq = jnp.where(is_gene, av[:, 0], c4)[:, None]

    bm = 1000
    h2 = pl.pallas_call(
        _fw_table_kernel,
        grid=(n // bm,),
        in_specs=[pl.BlockSpec((bm, f), lambda i: (i, 0)),
                  pl.BlockSpec((f, h_dim), lambda i: (0, 0)),
                  pl.BlockSpec((bm, 1), lambda i: (i, 0)),
                  pl.BlockSpec((bm, 1), lambda i: (i, 0))],
        out_specs=pl.BlockSpec((4, 2, bm, _QW), lambda i: (0, 0, i, 0)),
        out_shape=jax.ShapeDtypeStruct((4, 2, n, _QW), jnp.float32),
    )(features, W, so, av)
    h2 = h2.reshape(4 * 2 * n, _QW)

    srcp = jnp.pad(src.reshape(_NSUB, _E // _NSUB),
                   ((0, 0), (0, _ESUB - _E // _NSUB))).reshape(
                       _NSUB, _NCHUNK, _CW)
    dstp = jnp.pad(dst.reshape(_NSUB, _E // _NSUB),
                   ((0, 0), (0, _ESUB - _E // _NSUB)),
                   constant_values=-1).reshape(_NSUB, _NCHUNK, _CW)
    zeros = jnp.zeros((_SLAB, _QW), jnp.float32)

    gb = jnp.pad(is_gene, (0, _BITS_W * 32 - n)).reshape(_BITS_W, 32)
    bits = (gb.astype(jnp.uint32) << jnp.arange(32, dtype=jnp.uint32)
            ).sum(axis=1, dtype=jnp.uint32).astype(jnp.int32)

    acc = _edge_agg(h2, srcp, dstp, bits, zeros)

    z = pl.pallas_call(
        _combine_kernel,
        grid=(n // bm,),
        in_specs=[pl.BlockSpec((4, bm, _QW), lambda i: (0, i, 0)),
                  pl.BlockSpec((4, bm, _QW), lambda i: (0, i + _N // 1000, 0)),
                  pl.BlockSpec((bm, 1), lambda i: (i, 0)),
                  pl.BlockSpec((bm, 1), lambda i: (i, 0)),
                  pl.BlockSpec((bm, 1), lambda i: (i, 0)),
                  pl.BlockSpec((1, h_dim), lambda i: (0, 0))],
        out_specs=pl.BlockSpec((bm, h_dim), lambda i: (i, 0)),
        out_shape=jax.ShapeDtypeStruct((n, h_dim), jnp.float32),
    )(acc, acc, wp, wq, si, bias[None, :])

    bz = 1024
    adj = pl.pallas_call(
        _zzt_kernel,
        grid=(pl.cdiv(n, bz), pl.cdiv(n, bz)),
        in_specs=[pl.BlockSpec((bz, h_dim), lambda i, j: (i, 0)),
                  pl.BlockSpec((bz, h_dim), lambda i, j: (j, 0))],
        out_specs=pl.BlockSpec((bz, bz), lambda i, j: (i, j)),
        out_shape=jax.ShapeDtypeStruct((n, n), jnp.float32),
    )(z, z)
    return (adj, z)


# speedup vs baseline: 7.9085x; 1.4993x over previous
import functools

import jax
import jax.numpy as jnp
from jax import lax
from jax.experimental import pallas as pl
from jax.experimental.pallas import tpu as pltpu
from jax.experimental.pallas import tpu_sc as plsc

_N = 10000
_E = 160000
_NSUB = 16
_ESUB = 10240
_NCHUNK = 80
_CW = 128
_QW = 64
_ACC_ROWS = 20008
_NBUF = 3
_BITS_W = 320
_SLAB = 1248
_SLAB_EXTRA = 2 * _N - _NSUB * _SLAB


def _fw_table_kernel(f_ref, w_ref, so_ref, av_ref, o_ref):
    p = lax.dot_general(f_ref[...], w_ref[...], (((1,), (0,)), ((), ())),
                        preferred_element_type=jnp.float32)
    hb = p * so_ref[...]
    h1 = hb * av_ref[...]
    bm = hb.shape[0]
    hbq = hb.reshape(bm, 4, _QW).transpose(1, 0, 2)
    h1q = h1.reshape(bm, 4, _QW).transpose(1, 0, 2)
    o_ref[...] = jnp.stack([hbq, h1q], axis=1)


def _combine_kernel(ap_ref, aq_ref, wp_ref, wq_ref, si_ref, b_ref, z_ref):
    bm = ap_ref.shape[1]
    ap = ap_ref[...].transpose(1, 0, 2).reshape(bm, 4 * _QW)
    aq = aq_ref[...].transpose(1, 0, 2).reshape(bm, 4 * _QW)
    z_ref[...] = si_ref[...] * (wp_ref[...] * ap + wq_ref[...] * aq) + b_ref[...]


def _zzt_kernel(zi_ref, zj_ref, o_ref):
    o_ref[...] = lax.dot_general(
        zi_ref[...], zj_ref[...], (((1,), (1,)), ((), ())),
        preferred_element_type=jnp.float32)


_DUMP1 = 10008
_HROWS = 10016


def _degrees_alpha(srcp, dstp, node_ids, alpha_flat, zeros1d):
    mesh = plsc.VectorSubcoreMesh(core_axis_name="c", subcore_axis_name="s")
    nc2 = _NCHUNK // 2

    @functools.partial(
        pl.kernel,
        mesh=mesh,
        compiler_params=pltpu.CompilerParams(needs_layout_passes=False,
                                             use_tc_tiling_on_sc=False),
        out_type=(jax.ShapeDtypeStruct((2, 2, _HROWS), jnp.float32),
                  jax.ShapeDtypeStruct((_N,), jnp.float32)),
        scratch_types=[
            pltpu.VMEM((nc2, _CW), jnp.int32),
            pltpu.VMEM((nc2, _CW), jnp.int32),
            pltpu.VMEM((_N,), jnp.int32),
            pltpu.VMEM((2008,), jnp.float32),
            pltpu.VMEM((_CW,), jnp.float32),
            pltpu.VMEM((640,), jnp.float32),
            pltpu.VMEM_SHARED((_HROWS,), jnp.float32),
            pltpu.VMEM_SHARED((_HROWS,), jnp.float32),
            pltpu.SemaphoreType.DMA,
            pltpu.SemaphoreType.DMA,
        ],
    )
    def deg(src_hbm, dst_hbm, nid_hbm, alpha_hbm, zero_hbm, degs_hbm, av_hbm,
            hs_v, hd_v, node_v, alph_v, ones_v, av_v, sacc, dacc,
            sem0, sem1):
        cid = lax.axis_index("c")
        sid = lax.axis_index("s")

        pltpu.sync_copy(src_hbm.at[sid, pl.ds(cid * nc2, nc2)], hs_v)
        pltpu.sync_copy(dst_hbm.at[sid, pl.ds(cid * nc2, nc2)], hd_v)
        pltpu.sync_copy(nid_hbm, node_v)
        pltpu.sync_copy(alpha_hbm, alph_v)
        for k in range(_CW // 16):
            ones_v[pl.ds(k * 16, 16)] = jnp.full((16,), 1.0, jnp.float32)

        def hist_idx_body(j, carry):
            for k in range(_CW // 16):
                ds = pl.ds(k * 16, 16)
                s = hs_v[j, ds]
                d = hd_v[j, ds]
                pad = d < 0
                hs_v[j, ds] = jnp.where(pad, _DUMP1, s)
                hd_v[j, ds] = jnp.where(pad, _DUMP1, d)
            return carry

        lax.fori_loop(0, nc2, hist_idx_body, 0)

        pltpu.sync_copy(zero_hbm.at[pl.ds(0, 624)],
                        sacc.at[pl.ds(sid * 624, 624)])
        pltpu.sync_copy(zero_hbm.at[pl.ds(0, 624)],
                        dacc.at[pl.ds(sid * 624, 624)])

        @pl.when(sid == _NSUB - 1)
        def _():
            pltpu.sync_copy(zero_hbm.at[pl.ds(0, 32)],
                            sacc.at[pl.ds(624 * _NSUB, 32)])
            pltpu.sync_copy(zero_hbm.at[pl.ds(0, 32)],
                            dacc.at[pl.ds(624 * _NSUB, 32)])

        @pl.when(cid == 0)
        def _():
            nv = jnp.where(sid == _NSUB - 1, 25, 40)

            def av_body(j, carry):
                ds = pl.ds(j * 16, 16)
                nid = node_v[pl.ds(sid * 640 + j * 16, 16)]
                a = plsc.load_gather(alph_v, [jnp.maximum(nid, 0)])
                av_v[ds] = jnp.where(nid >= 0, a, 1.0)
                return carry

            lax.fori_loop(0, nv, av_body, 0)

            @pl.when(sid < _NSUB - 1)
            def _():
                pltpu.sync_copy(av_v, av_hbm.at[pl.ds(sid * 640, 640)])

            @pl.when(sid == _NSUB - 1)
            def _():
                pltpu.sync_copy(av_v.at[pl.ds(0, 400)],
                                av_hbm.at[pl.ds(sid * 640, 400)])

        plsc.subcore_barrier()

        handles = []
        for j in range(nc2):
            handles.append(pltpu.async_copy(
                ones_v, sacc.at[hs_v.at[j]], sem0, add=True))
            handles.append(pltpu.async_copy(
                ones_v, dacc.at[hd_v.at[j]], sem1, add=True))
        for h in handles:
            h.wait()

        plsc.subcore_barrier()

        @pl.when(sid == 0)
        def _():
            pltpu.sync_copy(sacc, degs_hbm.at[cid, 0])

        @pl.when(sid == 1)
        def _():
            pltpu.sync_copy(dacc, degs_hbm.at[cid, 1])

    return deg(srcp, dstp, node_ids, alpha_flat, zeros1d)


def _edge_agg(h2, srcp, dstp, bits, zeros):
    mesh = plsc.VectorSubcoreMesh(core_axis_name="c", subcore_axis_name="s")

    @functools.partial(
        pl.kernel,
        mesh=mesh,
        compiler_params=pltpu.CompilerParams(needs_layout_passes=False,
                                             use_tc_tiling_on_sc=False),
        out_type=jax.ShapeDtypeStruct((4, 2 * _N, _QW), jnp.float32),
        scratch_types=[
            pltpu.VMEM((_BITS_W,), jnp.int32),
            pltpu.VMEM((_NCHUNK, _CW), jnp.int32),
            pltpu.VMEM((_NCHUNK, _CW), jnp.int32),
            pltpu.VMEM((_NBUF, _CW, _QW), jnp.float32),
            pltpu.VMEM_SHARED((_ACC_ROWS, _QW), jnp.float32),
            [pltpu.SemaphoreType.DMA] * _NBUF,
            [pltpu.SemaphoreType.DMA] * _NBUF,
        ],
    )
    def agg(h2_hbm, src_hbm, dst_hbm, bits_hbm, zero_hbm, acc_hbm,
            bits_v, gidx_v, aidx_v, bufs_v, acc_sh, gsems, ssems):
        cid = lax.axis_index("c")
        sid = lax.axis_index("s")
        slab = sid * _SLAB

        pltpu.sync_copy(bits_hbm, bits_v)
        pltpu.sync_copy(src_hbm.at[sid], gidx_v)
        pltpu.sync_copy(dst_hbm.at[sid], aidx_v)

        base = cid * 2 * (2 * _N)

        def idx_body(j, carry):
            for k in range(_CW // 16):
                ds = pl.ds(k * 16, 16)
                s = gidx_v[j, ds]
                d = aidx_v[j, ds]
                dc = jnp.maximum(d, 0)
                sw = plsc.load_gather(bits_v, [lax.shift_right_logical(s, 5)])
                dw = plsc.load_gather(bits_v, [lax.shift_right_logical(dc, 5)])
                s_gene = lax.shift_right_logical(sw, s & 31) & 1
                d_gene = lax.shift_right_logical(dw, dc & 31) & 1
                t = (s_gene == 1) & (d_gene == 0)
                gidx_v[j, ds] = base + s + jnp.where(t, _N, 0)
                aidx_v[j, ds] = jnp.where(d < 0, 2 * _N,
                                          d + jnp.where(s_gene == 0, _N, 0))
            return carry

        def bump_body(j, carry):
            for k in range(_CW // 16):
                ds = pl.ds(k * 16, 16)
                gidx_v[j, ds] = gidx_v[j, ds] + 2 * _N
            return carry

        for q in range(2):
            qg = cid * 2 + q
            lax.fori_loop(0, _NCHUNK, idx_body if q == 0 else bump_body, 0)

            pltpu.sync_copy(zero_hbm, acc_sh.at[pl.ds(slab, _SLAB)])

            @pl.when(sid == _NSUB - 1)
            def _():
                pltpu.sync_copy(zero_hbm.at[pl.ds(0, _SLAB_EXTRA)],
                                acc_sh.at[pl.ds(_NSUB * _SLAB, _SLAB_EXTRA)])

            plsc.subcore_barrier()

            ghand = [None] * _NCHUNK
            shand = [None] * _NCHUNK

            def start_gather(j):
                return pltpu.async_copy(h2_hbm.at[gidx_v.at[j]],
                                        bufs_v.at[j % _NBUF],
                                        gsems[j % _NBUF])

            def start_scatter(j):
                return pltpu.async_copy(bufs_v.at[j % _NBUF],
                                        acc_sh.at[aidx_v.at[j]],
                                        ssems[j % _NBUF], add=True)

            for j in range(_NCHUNK):
                if j >= _NBUF:
                    shand[j - _NBUF].wait()
                ghand[j] = start_gather(j)
                if j >= 2:
                    ghand[j - 2].wait()
                    shand[j - 2] = start_scatter(j - 2)
            for j in range(_NCHUNK - 2, _NCHUNK):
                ghand[j].wait()
                shand[j] = start_scatter(j)
            for j in range(_NCHUNK - _NBUF, _NCHUNK):
                shand[j].wait()

            plsc.subcore_barrier()
            pltpu.sync_copy(acc_sh.at[pl.ds(slab, _SLAB)],
                            acc_hbm.at[qg, pl.ds(slab, _SLAB)])

            @pl.when(sid == _NSUB - 1)
            def _():
                pltpu.sync_copy(
                    acc_sh.at[pl.ds(_NSUB * _SLAB, _SLAB_EXTRA)],
                    acc_hbm.at[qg, pl.ds(_NSUB * _SLAB, _SLAB_EXTRA)])

    return agg(h2, srcp, dstp, bits, zeros)


def kernel(features, edge_index, node_ids, W, bias, alpha):
    n, f = features.shape
    h_dim = W.shape[1]
    gene_num = alpha.shape[0] - 2
    src = edge_index[0]
    dst = edge_index[1]

    srcp = jnp.pad(src.reshape(_NSUB, _E // _NSUB),
                   ((0, 0), (0, _ESUB - _E // _NSUB))).reshape(
                       _NSUB, _NCHUNK, _CW)
    dstp = jnp.pad(dst.reshape(_NSUB, _E // _NSUB),
                   ((0, 0), (0, _ESUB - _E // _NSUB)),
                   constant_values=-1).reshape(_NSUB, _NCHUNK, _CW)
    alpha_flat = jnp.pad(alpha[:, 0], (0, 2008 - alpha.shape[0]))
    zeros1d = jnp.zeros((640,), jnp.float32)

    degs, av1 = _degrees_alpha(srcp, dstp, node_ids, alpha_flat, zeros1d)
    out_deg = jnp.clip(degs[0, 0, :n] + degs[1, 0, :n], 1.0, None)
    in_deg = jnp.clip(degs[0, 1, :n] + degs[1, 1, :n], 1.0, None)
    so = (out_deg ** -0.5)[:, None]
    si = (in_deg ** -0.5)[:, None]

    is_gene = node_ids >= 0
    av = av1[:, None]
    c3 = alpha[gene_num, 0]
    c4 = alpha[gene_num + 1, 0]
    wp = jnp.where(is_gene, c3, 1.0)[:, None]
    wq = jnp.where(is_gene, av1, c4)[:, None]

    bm = 1000
    h2 = pl.pallas_call(
        _fw_table_kernel,
        grid=(n // bm,),
        in_specs=[pl.BlockSpec((bm, f), lambda i: (i, 0)),
                  pl.BlockSpec((f, h_dim), lambda i: (0, 0)),
                  pl.BlockSpec((bm, 1), lambda i: (i, 0)),
                  pl.BlockSpec((bm, 1), lambda i: (i, 0))],
        out_specs=pl.BlockSpec((4, 2, bm, _QW), lambda i: (0, 0, i, 0)),
        out_shape=jax.ShapeDtypeStruct((4, 2, n, _QW), jnp.float32),
    )(features, W, so, av)
    h2 = h2.reshape(4 * 2 * n, _QW)

    zeros = jnp.zeros((_SLAB, _QW), jnp.float32)

    gb = jnp.pad(is_gene, (0, _BITS_W * 32 - n)).reshape(_BITS_W, 32)
    bits = (gb.astype(jnp.uint32) << jnp.arange(32, dtype=jnp.uint32)
            ).sum(axis=1, dtype=jnp.uint32).astype(jnp.int32)

    acc = _edge_agg(h2, srcp, dstp, bits, zeros)

    z = pl.pallas_call(
        _combine_kernel,
        grid=(n // bm,),
        in_specs=[pl.BlockSpec((4, bm, _QW), lambda i: (0, i, 0)),
                  pl.BlockSpec((4, bm, _QW), lambda i: (0, i + _N // 1000, 0)),
                  pl.BlockSpec((bm, 1), lambda i: (i, 0)),
                  pl.BlockSpec((bm, 1), lambda i: (i, 0)),
                  pl.BlockSpec((bm, 1), lambda i: (i, 0)),
                  pl.BlockSpec((1, h_dim), lambda i: (0, 0))],
        out_specs=pl.BlockSpec((bm, h_dim), lambda i: (i, 0)),
        out_shape=jax.ShapeDtypeStruct((n, h_dim), jnp.float32),
    )(acc, acc, wp, wq, si, bias[None, :])

    bz = 1024
    adj = pl.pallas_call(
        _zzt_kernel,
        grid=(pl.cdiv(n, bz), pl.cdiv(n, bz)),
        in_specs=[pl.BlockSpec((bz, h_dim), lambda i, j: (i, 0)),
                  pl.BlockSpec((bz, h_dim), lambda i, j: (j, 0))],
        out_specs=pl.BlockSpec((bz, bz), lambda i, j: (i, j)),
        out_shape=jax.ShapeDtypeStruct((n, n), jnp.float32),
    )(z, z)
    return (adj, z)


# speedup vs baseline: 8.0308x; 1.0155x over previous
import functools

import jax
import jax.numpy as jnp
from jax import lax
from jax.experimental import pallas as pl
from jax.experimental.pallas import tpu as pltpu
from jax.experimental.pallas import tpu_sc as plsc

_N = 10000
_E = 160000
_NSUB = 16
_ESUB = 10240
_NCHUNK = 80
_CW = 128
_QW = 64
_ACC_ROWS = 20008
_NBUF = 3
_BITS_W = 320
_SLAB = 1248
_SLAB_EXTRA = 2 * _N - _NSUB * _SLAB


def _fw_table_kernel(f_ref, w_ref, so_ref, av_ref, o_ref):
    p = lax.dot_general(f_ref[...], w_ref[...], (((1,), (0,)), ((), ())),
                        preferred_element_type=jnp.float32)
    hb = p * so_ref[...]
    h1 = hb * av_ref[...]
    bm = hb.shape[0]
    hbq = hb.reshape(bm, 4, _QW).transpose(1, 0, 2)
    h1q = h1.reshape(bm, 4, _QW).transpose(1, 0, 2)
    o_ref[...] = jnp.stack([hbq, h1q], axis=1)


def _combine_kernel(ap_ref, aq_ref, wp_ref, wq_ref, si_ref, b_ref, z_ref):
    bm = ap_ref.shape[1]
    ap = ap_ref[...].transpose(1, 0, 2).reshape(bm, 4 * _QW)
    aq = aq_ref[...].transpose(1, 0, 2).reshape(bm, 4 * _QW)
    z_ref[...] = si_ref[...] * (wp_ref[...] * ap + wq_ref[...] * aq) + b_ref[...]


def _zzt_kernel(zi_ref, zj_ref, o_ref):
    o_ref[...] = lax.dot_general(
        zi_ref[...], zj_ref[...], (((1,), (1,)), ((), ())),
        preferred_element_type=jnp.float32)


_DUMP1 = 10008
_HROWS = 10016


def _degrees_alpha(srcp, dstp, node_ids, alpha_flat, zeros1d):
    mesh = plsc.VectorSubcoreMesh(core_axis_name="c", subcore_axis_name="s")
    nc2 = _NCHUNK // 2

    @functools.partial(
        pl.kernel,
        mesh=mesh,
        compiler_params=pltpu.CompilerParams(needs_layout_passes=False,
                                             use_tc_tiling_on_sc=False),
        out_type=(jax.ShapeDtypeStruct((2, 2, _HROWS), jnp.float32),
                  jax.ShapeDtypeStruct((_N,), jnp.float32)),
        scratch_types=[
            pltpu.VMEM((nc2, _CW), jnp.int32),
            pltpu.VMEM((nc2, _CW), jnp.int32),
            pltpu.VMEM((_N,), jnp.int32),
            pltpu.VMEM((2008,), jnp.float32),
            pltpu.VMEM((_CW,), jnp.float32),
            pltpu.VMEM((640,), jnp.float32),
            pltpu.VMEM_SHARED((_HROWS,), jnp.float32),
            pltpu.VMEM_SHARED((_HROWS,), jnp.float32),
            pltpu.SemaphoreType.DMA,
            pltpu.SemaphoreType.DMA,
        ],
    )
    def deg(src_hbm, dst_hbm, nid_hbm, alpha_hbm, zero_hbm, degs_hbm, av_hbm,
            hs_v, hd_v, node_v, alph_v, ones_v, av_v, sacc, dacc,
            sem0, sem1):
        cid = lax.axis_index("c")
        sid = lax.axis_index("s")

        pltpu.sync_copy(src_hbm.at[sid, pl.ds(cid * nc2, nc2)], hs_v)
        pltpu.sync_copy(dst_hbm.at[sid, pl.ds(cid * nc2, nc2)], hd_v)
        pltpu.sync_copy(nid_hbm, node_v)
        pltpu.sync_copy(alpha_hbm, alph_v)
        for k in range(_CW // 16):
            ones_v[pl.ds(k * 16, 16)] = jnp.full((16,), 1.0, jnp.float32)

        def hist_idx_body(j, carry):
            for k in range(_CW // 16):
                ds = pl.ds(k * 16, 16)
                s = hs_v[j, ds]
                d = hd_v[j, ds]
                pad = d < 0
                hs_v[j, ds] = jnp.where(pad, _DUMP1, s)
                hd_v[j, ds] = jnp.where(pad, _DUMP1, d)
            return carry

        lax.fori_loop(0, nc2, hist_idx_body, 0)

        pltpu.sync_copy(zero_hbm.at[pl.ds(0, 624)],
                        sacc.at[pl.ds(sid * 624, 624)])
        pltpu.sync_copy(zero_hbm.at[pl.ds(0, 624)],
                        dacc.at[pl.ds(sid * 624, 624)])

        @pl.when(sid == _NSUB - 1)
        def _():
            pltpu.sync_copy(zero_hbm.at[pl.ds(0, 32)],
                            sacc.at[pl.ds(624 * _NSUB, 32)])
            pltpu.sync_copy(zero_hbm.at[pl.ds(0, 32)],
                            dacc.at[pl.ds(624 * _NSUB, 32)])

        @pl.when(cid == 0)
        def _():
            nv = jnp.where(sid == _NSUB - 1, 25, 40)

            def av_body(j, carry):
                ds = pl.ds(j * 16, 16)
                nid = node_v[pl.ds(sid * 640 + j * 16, 16)]
                a = plsc.load_gather(alph_v, [jnp.maximum(nid, 0)])
                av_v[ds] = jnp.where(nid >= 0, a, 1.0)
                return carry

            lax.fori_loop(0, nv, av_body, 0)

            @pl.when(sid < _NSUB - 1)
            def _():
                pltpu.sync_copy(av_v, av_hbm.at[pl.ds(sid * 640, 640)])

            @pl.when(sid == _NSUB - 1)
            def _():
                pltpu.sync_copy(av_v.at[pl.ds(0, 400)],
                                av_hbm.at[pl.ds(sid * 640, 400)])

        plsc.subcore_barrier()

        handles = []
        for j in range(nc2):
            handles.append(pltpu.async_copy(
                ones_v, sacc.at[hs_v.at[j]], sem0, add=True))
            handles.append(pltpu.async_copy(
                ones_v, dacc.at[hd_v.at[j]], sem1, add=True))
        for h in handles:
            h.wait()

        plsc.subcore_barrier()

        @pl.when(sid == 0)
        def _():
            pltpu.sync_copy(sacc, degs_hbm.at[cid, 0])

        @pl.when(sid == 1)
        def _():
            pltpu.sync_copy(dacc, degs_hbm.at[cid, 1])

    return deg(srcp, dstp, node_ids, alpha_flat, zeros1d)


def _edge_agg(h2, srcp, dstp, bits, zeros):
    mesh = plsc.VectorSubcoreMesh(core_axis_name="c", subcore_axis_name="s")

    @functools.partial(
        pl.kernel,
        mesh=mesh,
        compiler_params=pltpu.CompilerParams(needs_layout_passes=False,
                                             use_tc_tiling_on_sc=False),
        out_type=jax.ShapeDtypeStruct((4, 2 * _N, _QW), jnp.float32),
        scratch_types=[
            pltpu.VMEM((_BITS_W,), jnp.int32),
            pltpu.VMEM((_NCHUNK, _CW), jnp.int32),
            pltpu.VMEM((_NCHUNK, _CW), jnp.int32),
            pltpu.VMEM((_NBUF, _CW, _QW), jnp.float32),
            pltpu.VMEM_SHARED((_ACC_ROWS, _QW), jnp.float32),
            [pltpu.SemaphoreType.DMA] * _NBUF,
            [pltpu.SemaphoreType.DMA] * _NBUF,
        ],
    )
    def agg(h2_hbm, src_hbm, dst_hbm, bits_hbm, zero_hbm, acc_hbm,
            bits_v, gidx_v, aidx_v, bufs_v, acc_sh, gsems, ssems):
        cid = lax.axis_index("c")
        sid = lax.axis_index("s")
        slab = sid * _SLAB

        pltpu.sync_copy(bits_hbm, bits_v)
        pltpu.sync_copy(src_hbm.at[sid], gidx_v)
        pltpu.sync_copy(dst_hbm.at[sid], aidx_v)

        base = cid * 2 * (2 * _N)

        def idx_body(j, carry):
            for k in range(_CW // 16):
                ds = pl.ds(k * 16, 16)
                s = gidx_v[j, ds]
                d = aidx_v[j, ds]
                dc = jnp.maximum(d, 0)
                sw = plsc.load_gather(bits_v, [lax.shift_right_logical(s, 5)])
                dw = plsc.load_gather(bits_v, [lax.shift_right_logical(dc, 5)])
                s_gene = lax.shift_right_logical(sw, s & 31) & 1
                d_gene = lax.shift_right_logical(dw, dc & 31) & 1
                t = (s_gene == 1) & (d_gene == 0)
                gidx_v[j, ds] = base + s + jnp.where(t, _N, 0)
                aidx_v[j, ds] = jnp.where(d < 0, 2 * _N,
                                          d + jnp.where(s_gene == 0, _N, 0))
            return carry

        def bump_body(j, carry):
            for k in range(_CW // 16):
                ds = pl.ds(k * 16, 16)
                gidx_v[j, ds] = gidx_v[j, ds] + 2 * _N
            return carry

        for q in range(2):
            qg = cid * 2 + q
            lax.fori_loop(0, _NCHUNK, idx_body if q == 0 else bump_body, 0)

            pltpu.sync_copy(zero_hbm, acc_sh.at[pl.ds(slab, _SLAB)])

            @pl.when(sid == _NSUB - 1)
            def _():
                pltpu.sync_copy(zero_hbm.at[pl.ds(0, _SLAB_EXTRA)],
                                acc_sh.at[pl.ds(_NSUB * _SLAB, _SLAB_EXTRA)])

            plsc.subcore_barrier()

            ghand = [None] * _NCHUNK
            shand = [None] * _NCHUNK

            def start_gather(j):
                return pltpu.async_copy(h2_hbm.at[gidx_v.at[j]],
                                        bufs_v.at[j % _NBUF],
                                        gsems[j % _NBUF])

            def start_scatter(j):
                return pltpu.async_copy(bufs_v.at[j % _NBUF],
                                        acc_sh.at[aidx_v.at[j]],
                                        ssems[j % _NBUF], add=True)

            for j in range(_NCHUNK):
                if j >= _NBUF:
                    shand[j - _NBUF].wait()
                ghand[j] = start_gather(j)
                if j >= 2:
                    ghand[j - 2].wait()
                    shand[j - 2] = start_scatter(j - 2)
            for j in range(_NCHUNK - 2, _NCHUNK):
                ghand[j].wait()
                shand[j] = start_scatter(j)
            for j in range(_NCHUNK - _NBUF, _NCHUNK):
                shand[j].wait()

            plsc.subcore_barrier()
            pltpu.sync_copy(acc_sh.at[pl.ds(slab, _SLAB)],
                            acc_hbm.at[qg, pl.ds(slab, _SLAB)])

            @pl.when(sid == _NSUB - 1)
            def _():
                pltpu.sync_copy(
                    acc_sh.at[pl.ds(_NSUB * _SLAB, _SLAB_EXTRA)],
                    acc_hbm.at[qg, pl.ds(_NSUB * _SLAB, _SLAB_EXTRA)])

    return agg(h2, srcp, dstp, bits, zeros)


def kernel(features, edge_index, node_ids, W, bias, alpha):
    n, f = features.shape
    h_dim = W.shape[1]
    gene_num = alpha.shape[0] - 2
    src = edge_index[0]
    dst = edge_index[1]

    srcp = jnp.pad(src.reshape(_NSUB, _E // _NSUB),
                   ((0, 0), (0, _ESUB - _E // _NSUB))).reshape(
                       _NSUB, _NCHUNK, _CW)
    dstp = jnp.pad(dst.reshape(_NSUB, _E // _NSUB),
                   ((0, 0), (0, _ESUB - _E // _NSUB)),
                   constant_values=-1).reshape(_NSUB, _NCHUNK, _CW)
    alpha_flat = jnp.pad(alpha[:, 0], (0, 2008 - alpha.shape[0]))
    zeros1d = jnp.zeros((640,), jnp.float32)

    degs, av1 = _degrees_alpha(srcp, dstp, node_ids, alpha_flat, zeros1d)
    out_deg = jnp.clip(degs[0, 0, :n] + degs[1, 0, :n], 1.0, None)
    in_deg = jnp.clip(degs[0, 1, :n] + degs[1, 1, :n], 1.0, None)
    so = (out_deg ** -0.5)[:, None]
    si = (in_deg ** -0.5)[:, None]

    is_gene = node_ids >= 0
    av = av1[:, None]
    c3 = alpha[gene_num, 0]
    c4 = alpha[gene_num + 1, 0]
    wp = jnp.where(is_gene, c3, 1.0)[:, None]
    wq = jnp.where(is_gene, av1, c4)[:, None]

    bm = 1000
    h2 = pl.pallas_call(
        _fw_table_kernel,
        grid=(n // bm,),
        in_specs=[pl.BlockSpec((bm, f), lambda i: (i, 0)),
                  pl.BlockSpec((f, h_dim), lambda i: (0, 0)),
                  pl.BlockSpec((bm, 1), lambda i: (i, 0)),
                  pl.BlockSpec((bm, 1), lambda i: (i, 0))],
        out_specs=pl.BlockSpec((4, 2, bm, _QW), lambda i: (0, 0, i, 0)),
        out_shape=jax.ShapeDtypeStruct((4, 2, n, _QW), jnp.float32),
    )(features, W, so, av)
    h2 = h2.reshape(4 * 2 * n, _QW)

    zeros = jnp.zeros((_SLAB, _QW), jnp.float32)

    gb = jnp.pad(is_gene, (0, _BITS_W * 32 - n)).reshape(_BITS_W, 32)
    bits = (gb.astype(jnp.uint32) << jnp.arange(32, dtype=jnp.uint32)
            ).sum(axis=1, dtype=jnp.uint32).astype(jnp.int32)

    acc = _edge_agg(h2, srcp, dstp, bits, zeros)

    z = pl.pallas_call(
        _combine_kernel,
        grid=(n // bm,),
        in_specs=[pl.BlockSpec((4, bm, _QW), lambda i: (0, i, 0)),
                  pl.BlockSpec((4, bm, _QW), lambda i: (0, i + _N // 1000, 0)),
                  pl.BlockSpec((bm, 1), lambda i: (i, 0)),
                  pl.BlockSpec((bm, 1), lambda i: (i, 0)),
                  pl.BlockSpec((bm, 1), lambda i: (i, 0)),
                  pl.BlockSpec((1, h_dim), lambda i: (0, 0))],
        out_specs=pl.BlockSpec((bm, h_dim), lambda i: (i, 0)),
        out_shape=jax.ShapeDtypeStruct((n, h_dim), jnp.float32),
    )(acc, acc, wp, wq, si, bias[None, :])

    zb = z.astype(jnp.bfloat16)
    bz = 1024
    adj = pl.pallas_call(
        _zzt_kernel,
        grid=(pl.cdiv(n, bz), pl.cdiv(n, bz)),
        in_specs=[pl.BlockSpec((bz, h_dim), lambda i, j: (i, 0)),
                  pl.BlockSpec((bz, h_dim), lambda i, j: (j, 0))],
        out_specs=pl.BlockSpec((bz, bz), lambda i, j: (i, j)),
        out_shape=jax.ShapeDtypeStruct((n, n), jnp.float32),
    )(zb, zb)
    return (adj, z)


# speedup vs baseline: 8.1026x; 1.0089x over previous
import functools

import jax
import jax.numpy as jnp
from jax import lax
from jax.experimental import pallas as pl
from jax.experimental.pallas import tpu as pltpu
from jax.experimental.pallas import tpu_sc as plsc

_N = 10000
_E = 160000
_NSUB = 16
_ESUB = 10240
_NCHUNK = 80
_CW = 128
_QW = 64
_ACC_ROWS = 20008
_NBUF = 3
_BITS_W = 320
_SLAB = 1248
_SLAB_EXTRA = 2 * _N - _NSUB * _SLAB


def _fw_table_kernel(f_ref, w_ref, so_ref, av_ref, o_ref):
    p = lax.dot_general(f_ref[...], w_ref[...], (((1,), (0,)), ((), ())),
                        preferred_element_type=jnp.float32)
    hb = p * so_ref[...]
    h1 = hb * av_ref[...]
    bm = hb.shape[0]
    hbq = hb.reshape(bm, 4, _QW).transpose(1, 0, 2)
    h1q = h1.reshape(bm, 4, _QW).transpose(1, 0, 2)
    o_ref[...] = jnp.stack([hbq, h1q], axis=1)


def _combine_kernel(ap_ref, aq_ref, wp_ref, wq_ref, si_ref, b_ref,
                    z_ref, zb_ref):
    bm = ap_ref.shape[1]
    ap = ap_ref[...].transpose(1, 0, 2).reshape(bm, 4 * _QW)
    aq = aq_ref[...].transpose(1, 0, 2).reshape(bm, 4 * _QW)
    z = si_ref[...] * (wp_ref[...] * ap + wq_ref[...] * aq) + b_ref[...]
    z_ref[...] = z
    zb_ref[...] = z.astype(jnp.bfloat16)


def _zzt_kernel(zi_ref, zj_ref, o_ref):
    o_ref[...] = lax.dot_general(
        zi_ref[...], zj_ref[...], (((1,), (1,)), ((), ())),
        preferred_element_type=jnp.float32)


_DUMP1 = 10008
_HROWS = 10016


def _degrees_alpha(srcp, dstp, node_ids, alpha_flat, zeros1d):
    mesh = plsc.VectorSubcoreMesh(core_axis_name="c", subcore_axis_name="s")
    nc2 = _NCHUNK // 2

    @functools.partial(
        pl.kernel,
        mesh=mesh,
        compiler_params=pltpu.CompilerParams(needs_layout_passes=False,
                                             use_tc_tiling_on_sc=False),
        out_type=(jax.ShapeDtypeStruct((2, 2, _HROWS), jnp.float32),
                  jax.ShapeDtypeStruct((_N,), jnp.float32)),
        scratch_types=[
            pltpu.VMEM((nc2, _CW), jnp.int32),
            pltpu.VMEM((nc2, _CW), jnp.int32),
            pltpu.VMEM((_N,), jnp.int32),
            pltpu.VMEM((2008,), jnp.float32),
            pltpu.VMEM((_CW,), jnp.float32),
            pltpu.VMEM((640,), jnp.float32),
            pltpu.VMEM_SHARED((_HROWS,), jnp.float32),
            pltpu.VMEM_SHARED((_HROWS,), jnp.float32),
            pltpu.SemaphoreType.DMA,
            pltpu.SemaphoreType.DMA,
        ],
    )
    def deg(src_hbm, dst_hbm, nid_hbm, alpha_hbm, zero_hbm, degs_hbm, av_hbm,
            hs_v, hd_v, node_v, alph_v, ones_v, av_v, sacc, dacc,
            sem0, sem1):
        cid = lax.axis_index("c")
        sid = lax.axis_index("s")

        pltpu.sync_copy(src_hbm.at[sid, pl.ds(cid * nc2, nc2)], hs_v)
        pltpu.sync_copy(dst_hbm.at[sid, pl.ds(cid * nc2, nc2)], hd_v)
        pltpu.sync_copy(nid_hbm, node_v)
        pltpu.sync_copy(alpha_hbm, alph_v)
        for k in range(_CW // 16):
            ones_v[pl.ds(k * 16, 16)] = jnp.full((16,), 1.0, jnp.float32)

        def hist_idx_body(j, carry):
            for k in range(_CW // 16):
                ds = pl.ds(k * 16, 16)
                s = hs_v[j, ds]
                d = hd_v[j, ds]
                pad = d < 0
                hs_v[j, ds] = jnp.where(pad, _DUMP1, s)
                hd_v[j, ds] = jnp.where(pad, _DUMP1, d)
            return carry

        lax.fori_loop(0, nc2, hist_idx_body, 0)

        pltpu.sync_copy(zero_hbm.at[pl.ds(0, 624)],
                        sacc.at[pl.ds(sid * 624, 624)])
        pltpu.sync_copy(zero_hbm.at[pl.ds(0, 624)],
                        dacc.at[pl.ds(sid * 624, 624)])

        @pl.when(sid == _NSUB - 1)
        def _():
            pltpu.sync_copy(zero_hbm.at[pl.ds(0, 32)],
                            sacc.at[pl.ds(624 * _NSUB, 32)])
            pltpu.sync_copy(zero_hbm.at[pl.ds(0, 32)],
                            dacc.at[pl.ds(624 * _NSUB, 32)])

        @pl.when(cid == 0)
        def _():
            nv = jnp.where(sid == _NSUB - 1, 25, 40)

            def av_body(j, carry):
                ds = pl.ds(j * 16, 16)
                nid = node_v[pl.ds(sid * 640 + j * 16, 16)]
                a = plsc.load_gather(alph_v, [jnp.maximum(nid, 0)])
                av_v[ds] = jnp.where(nid >= 0, a, 1.0)
                return carry

            lax.fori_loop(0, nv, av_body, 0)

            @pl.when(sid < _NSUB - 1)
            def _():
                pltpu.sync_copy(av_v, av_hbm.at[pl.ds(sid * 640, 640)])

            @pl.when(sid == _NSUB - 1)
            def _():
                pltpu.sync_copy(av_v.at[pl.ds(0, 400)],
                                av_hbm.at[pl.ds(sid * 640, 400)])

        plsc.subcore_barrier()

        handles = []
        for j in range(nc2):
            handles.append(pltpu.async_copy(
                ones_v, sacc.at[hs_v.at[j]], sem0, add=True))
            handles.append(pltpu.async_copy(
                ones_v, dacc.at[hd_v.at[j]], sem1, add=True))
        for h in handles:
            h.wait()

        plsc.subcore_barrier()

        @pl.when(sid == 0)
        def _():
            pltpu.sync_copy(sacc, degs_hbm.at[cid, 0])

        @pl.when(sid == 1)
        def _():
            pltpu.sync_copy(dacc, degs_hbm.at[cid, 1])

    return deg(srcp, dstp, node_ids, alpha_flat, zeros1d)


def _edge_agg(h2, srcp, dstp, bits, zeros):
    mesh = plsc.VectorSubcoreMesh(core_axis_name="c", subcore_axis_name="s")

    @functools.partial(
        pl.kernel,
        mesh=mesh,
        compiler_params=pltpu.CompilerParams(needs_layout_passes=False,
                                             use_tc_tiling_on_sc=False),
        out_type=jax.ShapeDtypeStruct((4, 2 * _N, _QW), jnp.float32),
        scratch_types=[
            pltpu.VMEM((_BITS_W,), jnp.int32),
            pltpu.VMEM((_NCHUNK, _CW), jnp.int32),
            pltpu.VMEM((_NCHUNK, _CW), jnp.int32),
            pltpu.VMEM((_NBUF, _CW, _QW), jnp.float32),
            pltpu.VMEM_SHARED((_ACC_ROWS, _QW), jnp.float32),
            [pltpu.SemaphoreType.DMA] * _NBUF,
            [pltpu.SemaphoreType.DMA] * _NBUF,
        ],
    )
    def agg(h2_hbm, src_hbm, dst_hbm, bits_hbm, zero_hbm, acc_hbm,
            bits_v, gidx_v, aidx_v, bufs_v, acc_sh, gsems, ssems):
        cid = lax.axis_index("c")
        sid = lax.axis_index("s")
        slab = sid * _SLAB

        pltpu.sync_copy(bits_hbm, bits_v)
        pltpu.sync_copy(src_hbm.at[sid], gidx_v)
        pltpu.sync_copy(dst_hbm.at[sid], aidx_v)

        base = cid * 2 * (2 * _N)

        def idx_body(j, carry):
            for k in range(_CW // 16):
                ds = pl.ds(k * 16, 16)
                s = gidx_v[j, ds]
                d = aidx_v[j, ds]
                dc = jnp.maximum(d, 0)
                sw = plsc.load_gather(bits_v, [lax.shift_right_logical(s, 5)])
                dw = plsc.load_gather(bits_v, [lax.shift_right_logical(dc, 5)])
                s_gene = lax.shift_right_logical(sw, s & 31) & 1
                d_gene = lax.shift_right_logical(dw, dc & 31) & 1
                t = (s_gene == 1) & (d_gene == 0)
                gidx_v[j, ds] = base + s + jnp.where(t, _N, 0)
                aidx_v[j, ds] = jnp.where(d < 0, 2 * _N,
                                          d + jnp.where(s_gene == 0, _N, 0))
            return carry

        def bump_body(j, carry):
            for k in range(_CW // 16):
                ds = pl.ds(k * 16, 16)
                gidx_v[j, ds] = gidx_v[j, ds] + 2 * _N
            return carry

        for q in range(2):
            qg = cid * 2 + q
            lax.fori_loop(0, _NCHUNK, idx_body if q == 0 else bump_body, 0)

            pltpu.sync_copy(zero_hbm, acc_sh.at[pl.ds(slab, _SLAB)])

            @pl.when(sid == _NSUB - 1)
            def _():
                pltpu.sync_copy(zero_hbm.at[pl.ds(0, _SLAB_EXTRA)],
                                acc_sh.at[pl.ds(_NSUB * _SLAB, _SLAB_EXTRA)])

            plsc.subcore_barrier()

            ghand = [None] * _NCHUNK
            shand = [None] * _NCHUNK

            def start_gather(j):
                return pltpu.async_copy(h2_hbm.at[gidx_v.at[j]],
                                        bufs_v.at[j % _NBUF],
                                        gsems[j % _NBUF])

            def start_scatter(j):
                return pltpu.async_copy(bufs_v.at[j % _NBUF],
                                        acc_sh.at[aidx_v.at[j]],
                                        ssems[j % _NBUF], add=True)

            for j in range(_NCHUNK):
                if j >= _NBUF:
                    shand[j - _NBUF].wait()
                ghand[j] = start_gather(j)
                if j >= 2:
                    ghand[j - 2].wait()
                    shand[j - 2] = start_scatter(j - 2)
            for j in range(_NCHUNK - 2, _NCHUNK):
                ghand[j].wait()
                shand[j] = start_scatter(j)
            for j in range(_NCHUNK - _NBUF, _NCHUNK):
                shand[j].wait()

            plsc.subcore_barrier()
            pltpu.sync_copy(acc_sh.at[pl.ds(slab, _SLAB)],
                            acc_hbm.at[qg, pl.ds(slab, _SLAB)])

            @pl.when(sid == _NSUB - 1)
            def _():
                pltpu.sync_copy(
                    acc_sh.at[pl.ds(_NSUB * _SLAB, _SLAB_EXTRA)],
                    acc_hbm.at[qg, pl.ds(_NSUB * _SLAB, _SLAB_EXTRA)])

    return agg(h2, srcp, dstp, bits, zeros)


def kernel(features, edge_index, node_ids, W, bias, alpha):
    n, f = features.shape
    h_dim = W.shape[1]
    gene_num = alpha.shape[0] - 2
    src = edge_index[0]
    dst = edge_index[1]

    srcp = jnp.pad(src.reshape(_NSUB, _E // _NSUB),
                   ((0, 0), (0, _ESUB - _E // _NSUB))).reshape(
                       _NSUB, _NCHUNK, _CW)
    dstp = jnp.pad(dst.reshape(_NSUB, _E // _NSUB),
                   ((0, 0), (0, _ESUB - _E // _NSUB)),
                   constant_values=-1).reshape(_NSUB, _NCHUNK, _CW)
    alpha_flat = jnp.pad(alpha[:, 0], (0, 2008 - alpha.shape[0]))
    zeros1d = jnp.zeros((640,), jnp.float32)

    degs, av1 = _degrees_alpha(srcp, dstp, node_ids, alpha_flat, zeros1d)
    out_deg = jnp.clip(degs[0, 0, :n] + degs[1, 0, :n], 1.0, None)
    in_deg = jnp.clip(degs[0, 1, :n] + degs[1, 1, :n], 1.0, None)
    so = (out_deg ** -0.5)[:, None]
    si = (in_deg ** -0.5)[:, None]

    is_gene = node_ids >= 0
    av = av1[:, None]
    c3 = alpha[gene_num, 0]
    c4 = alpha[gene_num + 1, 0]
    wp = jnp.where(is_gene, c3, 1.0)[:, None]
    wq = jnp.where(is_gene, av1, c4)[:, None]

    bm = 1000
    h2 = pl.pallas_call(
        _fw_table_kernel,
        grid=(n // bm,),
        in_specs=[pl.BlockSpec((bm, f), lambda i: (i, 0)),
                  pl.BlockSpec((f, h_dim), lambda i: (0, 0)),
                  pl.BlockSpec((bm, 1), lambda i: (i, 0)),
                  pl.BlockSpec((bm, 1), lambda i: (i, 0))],
        out_specs=pl.BlockSpec((4, 2, bm, _QW), lambda i: (0, 0, i, 0)),
        out_shape=jax.ShapeDtypeStruct((4, 2, n, _QW), jnp.float32),
    )(features, W, so, av)
    h2 = h2.reshape(4 * 2 * n, _QW)

    zeros = jnp.zeros((_SLAB, _QW), jnp.float32)

    gb = jnp.pad(is_gene, (0, _BITS_W * 32 - n)).reshape(_BITS_W, 32)
    bits = (gb.astype(jnp.uint32) << jnp.arange(32, dtype=jnp.uint32)
            ).sum(axis=1, dtype=jnp.uint32).astype(jnp.int32)

    acc = _edge_agg(h2, srcp, dstp, bits, zeros)

    z, zb = pl.pallas_call(
        _combine_kernel,
        grid=(n // bm,),
        in_specs=[pl.BlockSpec((4, bm, _QW), lambda i: (0, i, 0)),
                  pl.BlockSpec((4, bm, _QW), lambda i: (0, i + _N // 1000, 0)),
                  pl.BlockSpec((bm, 1), lambda i: (i, 0)),
                  pl.BlockSpec((bm, 1), lambda i: (i, 0)),
                  pl.BlockSpec((bm, 1), lambda i: (i, 0)),
                  pl.BlockSpec((1, h_dim), lambda i: (0, 0))],
        out_specs=[pl.BlockSpec((bm, h_dim), lambda i: (i, 0)),
                   pl.BlockSpec((bm, h_dim), lambda i: (i, 0))],
        out_shape=[jax.ShapeDtypeStruct((n, h_dim), jnp.float32),
                   jax.ShapeDtypeStruct((n, h_dim), jnp.bfloat16)],
    )(acc, acc, wp, wq, si, bias[None, :])

    bz = 1024
    adj = pl.pallas_call(
        _zzt_kernel,
        grid=(pl.cdiv(n, bz), pl.cdiv(n, bz)),
        in_specs=[pl.BlockSpec((bz, h_dim), lambda i, j: (i, 0)),
                  pl.BlockSpec((bz, h_dim), lambda i, j: (j, 0))],
        out_specs=pl.BlockSpec((bz, bz), lambda i, j: (i, j)),
        out_shape=jax.ShapeDtypeStruct((n, n), jnp.float32),
    )(zb, zb)
    return (adj, z)


# speedup vs baseline: 8.6258x; 1.0646x over previous
import functools

import jax
import jax.numpy as jnp
from jax import lax
from jax.experimental import pallas as pl
from jax.experimental.pallas import tpu as pltpu
from jax.experimental.pallas import tpu_sc as plsc

_N = 10000
_E = 160000
_NSUB = 16
_ESUB = 10240
_NCHUNK = 80
_CW = 128
_QW = 64
_ACC_ROWS = 20008
_NBUF = 3
_BITS_W = 320
_SLAB = 1248
_SLAB_EXTRA = 2 * _N - _NSUB * _SLAB


def _fw_table_kernel(f_ref, w_ref, so_ref, av_ref, o_ref):
    p = lax.dot_general(f_ref[...], w_ref[...], (((1,), (0,)), ((), ())),
                        preferred_element_type=jnp.float32)
    hb = p * so_ref[...]
    h1 = hb * av_ref[...]
    bm = hb.shape[0]
    hbq = hb.reshape(bm, 4, _QW).transpose(1, 0, 2)
    h1q = h1.reshape(bm, 4, _QW).transpose(1, 0, 2)
    o_ref[...] = jnp.stack([hbq, h1q], axis=1)


def _combine_kernel(ap_ref, aq_ref, wp_ref, wq_ref, si_ref, b_ref,
                    z_ref, zb_ref):
    bm = ap_ref.shape[1]
    ap = ap_ref[...].transpose(1, 0, 2).reshape(bm, 4 * _QW)
    aq = aq_ref[...].transpose(1, 0, 2).reshape(bm, 4 * _QW)
    z = si_ref[...] * (wp_ref[...] * ap + wq_ref[...] * aq) + b_ref[...]
    z_ref[...] = z
    zb_ref[...] = z.astype(jnp.bfloat16)


def _zzt_kernel(zi_ref, zj_ref, o_ref):
    o_ref[...] = lax.dot_general(
        zi_ref[...], zj_ref[...], (((1,), (1,)), ((), ())),
        preferred_element_type=jnp.float32)


_DUMP1 = 10008
_HROWS = 10016


def _degrees_alpha(srcp, dstp, node_ids, alpha_flat, zeros1d):
    mesh = plsc.VectorSubcoreMesh(core_axis_name="c", subcore_axis_name="s")
    nc2 = _NCHUNK // 2

    @functools.partial(
        pl.kernel,
        mesh=mesh,
        compiler_params=pltpu.CompilerParams(needs_layout_passes=False,
                                             use_tc_tiling_on_sc=False),
        out_type=(jax.ShapeDtypeStruct((2, 2, _HROWS), jnp.float32),
                  jax.ShapeDtypeStruct((_N,), jnp.float32)),
        scratch_types=[
            pltpu.VMEM((nc2, _CW), jnp.int32),
            pltpu.VMEM((nc2, _CW), jnp.int32),
            pltpu.VMEM((_N,), jnp.int32),
            pltpu.VMEM((2008,), jnp.float32),
            pltpu.VMEM((_CW,), jnp.float32),
            pltpu.VMEM((640,), jnp.float32),
            pltpu.VMEM_SHARED((_HROWS,), jnp.float32),
            pltpu.VMEM_SHARED((_HROWS,), jnp.float32),
            pltpu.SemaphoreType.DMA,
            pltpu.SemaphoreType.DMA,
        ],
    )
    def deg(src_hbm, dst_hbm, nid_hbm, alpha_hbm, zero_hbm, degs_hbm, av_hbm,
            hs_v, hd_v, node_v, alph_v, ones_v, av_v, sacc, dacc,
            sem0, sem1):
        cid = lax.axis_index("c")
        sid = lax.axis_index("s")

        pltpu.sync_copy(src_hbm.at[sid, pl.ds(cid * nc2, nc2)], hs_v)
        pltpu.sync_copy(dst_hbm.at[sid, pl.ds(cid * nc2, nc2)], hd_v)
        pltpu.sync_copy(nid_hbm, node_v)
        pltpu.sync_copy(alpha_hbm, alph_v)
        for k in range(_CW // 16):
            ones_v[pl.ds(k * 16, 16)] = jnp.full((16,), 1.0, jnp.float32)

        def hist_idx_body(j, carry):
            for k in range(_CW // 16):
                ds = pl.ds(k * 16, 16)
                s = hs_v[j, ds]
                d = hd_v[j, ds]
                pad = d < 0
                hs_v[j, ds] = jnp.where(pad, _DUMP1, s)
                hd_v[j, ds] = jnp.where(pad, _DUMP1, d)
            return carry

        lax.fori_loop(0, nc2, hist_idx_body, 0)

        pltpu.sync_copy(zero_hbm.at[pl.ds(0, 624)],
                        sacc.at[pl.ds(sid * 624, 624)])
        pltpu.sync_copy(zero_hbm.at[pl.ds(0, 624)],
                        dacc.at[pl.ds(sid * 624, 624)])

        @pl.when(sid == _NSUB - 1)
        def _():
            pltpu.sync_copy(zero_hbm.at[pl.ds(0, 32)],
                            sacc.at[pl.ds(624 * _NSUB, 32)])
            pltpu.sync_copy(zero_hbm.at[pl.ds(0, 32)],
                            dacc.at[pl.ds(624 * _NSUB, 32)])

        @pl.when(cid == 0)
        def _():
            nv = jnp.where(sid == _NSUB - 1, 25, 40)

            def av_body(j, carry):
                ds = pl.ds(j * 16, 16)
                nid = node_v[pl.ds(sid * 640 + j * 16, 16)]
                a = plsc.load_gather(alph_v, [jnp.maximum(nid, 0)])
                av_v[ds] = jnp.where(nid >= 0, a, 1.0)
                return carry

            lax.fori_loop(0, nv, av_body, 0)

            @pl.when(sid < _NSUB - 1)
            def _():
                pltpu.sync_copy(av_v, av_hbm.at[pl.ds(sid * 640, 640)])

            @pl.when(sid == _NSUB - 1)
            def _():
                pltpu.sync_copy(av_v.at[pl.ds(0, 400)],
                                av_hbm.at[pl.ds(sid * 640, 400)])

        plsc.subcore_barrier()

        handles = []
        for j in range(nc2):
            handles.append(pltpu.async_copy(
                ones_v, sacc.at[hs_v.at[j]], sem0, add=True))
            handles.append(pltpu.async_copy(
                ones_v, dacc.at[hd_v.at[j]], sem1, add=True))
        for h in handles:
            h.wait()

        plsc.subcore_barrier()

        @pl.when(sid == 0)
        def _():
            pltpu.sync_copy(sacc, degs_hbm.at[cid, 0])

        @pl.when(sid == 1)
        def _():
            pltpu.sync_copy(dacc, degs_hbm.at[cid, 1])

    return deg(srcp, dstp, node_ids, alpha_flat, zeros1d)


def _edge_agg(h2, srcp, dstp, bits, zeros):
    mesh = plsc.VectorSubcoreMesh(core_axis_name="c", subcore_axis_name="s")

    @functools.partial(
        pl.kernel,
        mesh=mesh,
        compiler_params=pltpu.CompilerParams(needs_layout_passes=False,
                                             use_tc_tiling_on_sc=False),
        out_type=jax.ShapeDtypeStruct((4, 2 * _N, _QW), jnp.float32),
        scratch_types=[
            pltpu.VMEM((_BITS_W,), jnp.int32),
            pltpu.VMEM((_NCHUNK, _CW), jnp.int32),
            pltpu.VMEM((_NCHUNK, _CW), jnp.int32),
            pltpu.VMEM((_NBUF, _CW, _QW), jnp.float32),
            pltpu.VMEM_SHARED((_ACC_ROWS, _QW), jnp.float32),
            [pltpu.SemaphoreType.DMA] * _NBUF,
            [pltpu.SemaphoreType.DMA] * _NBUF,
        ],
    )
    def agg(h2_hbm, src_hbm, dst_hbm, bits_hbm, zero_hbm, acc_hbm,
            bits_v, gidx_v, aidx_v, bufs_v, acc_sh, gsems, ssems):
        cid = lax.axis_index("c")
        sid = lax.axis_index("s")
        slab = sid * _SLAB

        pltpu.sync_copy(bits_hbm, bits_v)
        pltpu.sync_copy(src_hbm.at[sid], gidx_v)
        pltpu.sync_copy(dst_hbm.at[sid], aidx_v)

        base = cid * 2 * (2 * _N)

        def idx_body(j, carry):
            for k in range(_CW // 16):
                ds = pl.ds(k * 16, 16)
                s = gidx_v[j, ds]
                d = aidx_v[j, ds]
                dc = jnp.maximum(d, 0)
                sw = plsc.load_gather(bits_v, [lax.shift_right_logical(s, 5)])
                dw = plsc.load_gather(bits_v, [lax.shift_right_logical(dc, 5)])
                s_gene = lax.shift_right_logical(sw, s & 31) & 1
                d_gene = lax.shift_right_logical(dw, dc & 31) & 1
                t = (s_gene == 1) & (d_gene == 0)
                gidx_v[j, ds] = base + s + jnp.where(t, _N, 0)
                aidx_v[j, ds] = jnp.where(d < 0, 2 * _N,
                                          d + jnp.where(s_gene == 0, _N, 0))
            return carry

        def bump_body(j, carry):
            for k in range(_CW // 16):
                ds = pl.ds(k * 16, 16)
                gidx_v[j, ds] = gidx_v[j, ds] + 2 * _N
            return carry

        for q in range(2):
            qg = cid * 2 + q
            lax.fori_loop(0, _NCHUNK, idx_body if q == 0 else bump_body, 0)

            pltpu.sync_copy(zero_hbm, acc_sh.at[pl.ds(slab, _SLAB)])

            @pl.when(sid == _NSUB - 1)
            def _():
                pltpu.sync_copy(zero_hbm.at[pl.ds(0, _SLAB_EXTRA)],
                                acc_sh.at[pl.ds(_NSUB * _SLAB, _SLAB_EXTRA)])

            plsc.subcore_barrier()

            ghand = [None] * _NCHUNK
            shand = [None] * _NCHUNK

            def start_gather(j):
                return pltpu.async_copy(h2_hbm.at[gidx_v.at[j]],
                                        bufs_v.at[j % _NBUF],
                                        gsems[j % _NBUF])

            def start_scatter(j):
                return pltpu.async_copy(bufs_v.at[j % _NBUF],
                                        acc_sh.at[aidx_v.at[j]],
                                        ssems[j % _NBUF], add=True)

            for j in range(_NCHUNK):
                if j >= _NBUF:
                    shand[j - _NBUF].wait()
                ghand[j] = start_gather(j)
                if j >= 2:
                    ghand[j - 2].wait()
                    shand[j - 2] = start_scatter(j - 2)
            for j in range(_NCHUNK - 2, _NCHUNK):
                ghand[j].wait()
                shand[j] = start_scatter(j)
            for j in range(_NCHUNK - _NBUF, _NCHUNK):
                shand[j].wait()

            plsc.subcore_barrier()
            pltpu.sync_copy(acc_sh.at[pl.ds(slab, _SLAB)],
                            acc_hbm.at[qg, pl.ds(slab, _SLAB)])

            @pl.when(sid == _NSUB - 1)
            def _():
                pltpu.sync_copy(
                    acc_sh.at[pl.ds(_NSUB * _SLAB, _SLAB_EXTRA)],
                    acc_hbm.at[qg, pl.ds(_NSUB * _SLAB, _SLAB_EXTRA)])

    return agg(h2, srcp, dstp, bits, zeros)


def kernel(features, edge_index, node_ids, W, bias, alpha):
    n, f = features.shape
    h_dim = W.shape[1]
    gene_num = alpha.shape[0] - 2
    src = edge_index[0]
    dst = edge_index[1]

    srcp = jnp.pad(src.reshape(_NSUB, _E // _NSUB),
                   ((0, 0), (0, _ESUB - _E // _NSUB))).reshape(
                       _NSUB, _NCHUNK, _CW)
    dstp = jnp.pad(dst.reshape(_NSUB, _E // _NSUB),
                   ((0, 0), (0, _ESUB - _E // _NSUB)),
                   constant_values=-1).reshape(_NSUB, _NCHUNK, _CW)
    alpha_flat = jnp.pad(alpha[:, 0], (0, 2008 - alpha.shape[0]))
    zeros1d = jnp.zeros((640,), jnp.float32)

    degs, av1 = _degrees_alpha(srcp, dstp, node_ids, alpha_flat, zeros1d)
    out_deg = jnp.clip(degs[0, 0, :n] + degs[1, 0, :n], 1.0, None)
    in_deg = jnp.clip(degs[0, 1, :n] + degs[1, 1, :n], 1.0, None)
    so = (out_deg ** -0.5)[:, None]
    si = (in_deg ** -0.5)[:, None]

    is_gene = node_ids >= 0
    av = av1[:, None]
    c3 = alpha[gene_num, 0]
    c4 = alpha[gene_num + 1, 0]
    wp = jnp.where(is_gene, c3, 1.0)[:, None]
    wq = jnp.where(is_gene, av1, c4)[:, None]

    bm = 1000
    h2 = pl.pallas_call(
        _fw_table_kernel,
        grid=(n // bm,),
        in_specs=[pl.BlockSpec((bm, f), lambda i: (i, 0)),
                  pl.BlockSpec((f, h_dim), lambda i: (0, 0)),
                  pl.BlockSpec((bm, 1), lambda i: (i, 0)),
                  pl.BlockSpec((bm, 1), lambda i: (i, 0))],
        out_specs=pl.BlockSpec((4, 2, bm, _QW), lambda i: (0, 0, i, 0)),
        out_shape=jax.ShapeDtypeStruct((4, 2, n, _QW), jnp.float32),
    )(features, W, so, av)
    h2 = h2.reshape(4 * 2 * n, _QW)

    zeros = jnp.zeros((_SLAB, _QW), jnp.float32)

    gb = jnp.pad(is_gene, (0, _BITS_W * 32 - n)).reshape(_BITS_W, 32)
    bits = (gb.astype(jnp.uint32) << jnp.arange(32, dtype=jnp.uint32)
            ).sum(axis=1, dtype=jnp.uint32).astype(jnp.int32)

    acc = _edge_agg(h2, srcp, dstp, bits, zeros)

    z, zb = pl.pallas_call(
        _combine_kernel,
        grid=(n // bm,),
        in_specs=[pl.BlockSpec((4, bm, _QW), lambda i: (0, i, 0)),
                  pl.BlockSpec((4, bm, _QW), lambda i: (0, i + _N // 1000, 0)),
                  pl.BlockSpec((bm, 1), lambda i: (i, 0)),
                  pl.BlockSpec((bm, 1), lambda i: (i, 0)),
                  pl.BlockSpec((bm, 1), lambda i: (i, 0)),
                  pl.BlockSpec((1, h_dim), lambda i: (0, 0))],
        out_specs=[pl.BlockSpec((bm, h_dim), lambda i: (i, 0)),
                   pl.BlockSpec((bm, h_dim), lambda i: (i, 0))],
        out_shape=[jax.ShapeDtypeStruct((n, h_dim), jnp.float32),
                   jax.ShapeDtypeStruct((n, h_dim), jnp.bfloat16)],
    )(acc, acc, wp, wq, si, bias[None, :])

    bz = 2048
    adj = pl.pallas_call(
        _zzt_kernel,
        grid=(pl.cdiv(n, bz), pl.cdiv(n, bz)),
        in_specs=[pl.BlockSpec((bz, h_dim), lambda i, j: (i, 0)),
                  pl.BlockSpec((bz, h_dim), lambda i, j: (j, 0))],
        out_specs=pl.BlockSpec((bz, bz), lambda i, j: (i, j)),
        out_shape=jax.ShapeDtypeStruct((n, n), jnp.float32),
    )(zb, zb)
    return (adj, z)


# speedup vs baseline: 8.6492x; 1.0027x over previous
import functools

import jax
import jax.numpy as jnp
from jax import lax
from jax.experimental import pallas as pl
from jax.experimental.pallas import tpu as pltpu
from jax.experimental.pallas import tpu_sc as plsc

_N = 10000
_E = 160000
_NSUB = 16
_ESUB = 10240
_NCHUNK = 80
_CW = 128
_QW = 64
_ACC_ROWS = 20008
_NBUF = 3
_BITS_W = 320
_SLAB = 1248
_SLAB_EXTRA = 2 * _N - _NSUB * _SLAB


def _fw_table_kernel(f_ref, w_ref, so_ref, av_ref, o_ref):
    p = lax.dot_general(f_ref[...], w_ref[...], (((1,), (0,)), ((), ())),
                        preferred_element_type=jnp.float32)
    hb = p * so_ref[...]
    h1 = hb * av_ref[...]
    bm = hb.shape[0]
    hbq = hb.reshape(bm, 4, _QW).transpose(1, 0, 2)
    h1q = h1.reshape(bm, 4, _QW).transpose(1, 0, 2)
    o_ref[...] = jnp.stack([hbq, h1q], axis=1)


def _combine_kernel(ap_ref, aq_ref, wp_ref, wq_ref, si_ref, b_ref,
                    z_ref, zb_ref):
    bm = ap_ref.shape[1]
    ap = ap_ref[...].transpose(1, 0, 2).reshape(bm, 4 * _QW)
    aq = aq_ref[...].transpose(1, 0, 2).reshape(bm, 4 * _QW)
    z = si_ref[...] * (wp_ref[...] * ap + wq_ref[...] * aq) + b_ref[...]
    z_ref[...] = z
    zb_ref[...] = z.astype(jnp.bfloat16)


def _zzt_kernel(zi_ref, zj_ref, o_ref):
    o_ref[...] = lax.dot_general(
        zi_ref[...], zj_ref[...], (((1,), (1,)), ((), ())),
        preferred_element_type=jnp.float32)


_DUMP1 = 10008
_HROWS = 10016


def _degrees_alpha(srcp, dstp, node_ids, alpha_flat, zeros1d):
    mesh = plsc.VectorSubcoreMesh(core_axis_name="c", subcore_axis_name="s")
    nc2 = _NCHUNK // 2

    @functools.partial(
        pl.kernel,
        mesh=mesh,
        compiler_params=pltpu.CompilerParams(needs_layout_passes=False,
                                             use_tc_tiling_on_sc=False),
        out_type=(jax.ShapeDtypeStruct((2, 2, _HROWS), jnp.float32),
                  jax.ShapeDtypeStruct((_N,), jnp.float32)),
        scratch_types=[
            pltpu.VMEM((nc2, _CW), jnp.int32),
            pltpu.VMEM((nc2, _CW), jnp.int32),
            pltpu.VMEM((_N,), jnp.int32),
            pltpu.VMEM((2008,), jnp.float32),
            pltpu.VMEM((_CW,), jnp.float32),
            pltpu.VMEM((640,), jnp.float32),
            pltpu.VMEM_SHARED((_HROWS,), jnp.float32),
            pltpu.VMEM_SHARED((_HROWS,), jnp.float32),
            pltpu.SemaphoreType.DMA,
            pltpu.SemaphoreType.DMA,
        ],
    )
    def deg(src_hbm, dst_hbm, nid_hbm, alpha_hbm, zero_hbm, degs_hbm, av_hbm,
            hs_v, hd_v, node_v, alph_v, ones_v, av_v, sacc, dacc,
            sem0, sem1):
        cid = lax.axis_index("c")
        sid = lax.axis_index("s")

        pltpu.sync_copy(src_hbm.at[sid, pl.ds(cid * nc2, nc2)], hs_v)
        pltpu.sync_copy(dst_hbm.at[sid, pl.ds(cid * nc2, nc2)], hd_v)
        pltpu.sync_copy(nid_hbm, node_v)
        pltpu.sync_copy(alpha_hbm, alph_v)
        for k in range(_CW // 16):
            ones_v[pl.ds(k * 16, 16)] = jnp.full((16,), 1.0, jnp.float32)

        def hist_idx_body(j, carry):
            for k in range(_CW // 16):
                ds = pl.ds(k * 16, 16)
                s = hs_v[j, ds]
                d = hd_v[j, ds]
                pad = d < 0
                hs_v[j, ds] = jnp.where(pad, _DUMP1, s)
                hd_v[j, ds] = jnp.where(pad, _DUMP1, d)
            return carry

        lax.fori_loop(0, nc2, hist_idx_body, 0)

        pltpu.sync_copy(zero_hbm.at[pl.ds(0, 624)],
                        sacc.at[pl.ds(sid * 624, 624)])
        pltpu.sync_copy(zero_hbm.at[pl.ds(0, 624)],
                        dacc.at[pl.ds(sid * 624, 624)])

        @pl.when(sid == _NSUB - 1)
        def _():
            pltpu.sync_copy(zero_hbm.at[pl.ds(0, 32)],
                            sacc.at[pl.ds(624 * _NSUB, 32)])
            pltpu.sync_copy(zero_hbm.at[pl.ds(0, 32)],
                            dacc.at[pl.ds(624 * _NSUB, 32)])

        @pl.when(cid == 0)
        def _():
            nv = jnp.where(sid == _NSUB - 1, 25, 40)

            def av_body(j, carry):
                ds = pl.ds(j * 16, 16)
                nid = node_v[pl.ds(sid * 640 + j * 16, 16)]
                a = plsc.load_gather(alph_v, [jnp.maximum(nid, 0)])
                av_v[ds] = jnp.where(nid >= 0, a, 1.0)
                return carry

            lax.fori_loop(0, nv, av_body, 0)

            @pl.when(sid < _NSUB - 1)
            def _():
                pltpu.sync_copy(av_v, av_hbm.at[pl.ds(sid * 640, 640)])

            @pl.when(sid == _NSUB - 1)
            def _():
                pltpu.sync_copy(av_v.at[pl.ds(0, 400)],
                                av_hbm.at[pl.ds(sid * 640, 400)])

        plsc.subcore_barrier()

        handles = []
        for j in range(nc2):
            handles.append(pltpu.async_copy(
                ones_v, sacc.at[hs_v.at[j]], sem0, add=True))
            handles.append(pltpu.async_copy(
                ones_v, dacc.at[hd_v.at[j]], sem1, add=True))
        for h in handles:
            h.wait()

        plsc.subcore_barrier()

        @pl.when(sid == 0)
        def _():
            pltpu.sync_copy(sacc, degs_hbm.at[cid, 0])

        @pl.when(sid == 1)
        def _():
            pltpu.sync_copy(dacc, degs_hbm.at[cid, 1])

    return deg(srcp, dstp, node_ids, alpha_flat, zeros1d)


def _edge_agg(h2, srcp, dstp, bits, zeros):
    mesh = plsc.VectorSubcoreMesh(core_axis_name="c", subcore_axis_name="s")

    @functools.partial(
        pl.kernel,
        mesh=mesh,
        compiler_params=pltpu.CompilerParams(needs_layout_passes=False,
                                             use_tc_tiling_on_sc=False),
        out_type=jax.ShapeDtypeStruct((4, 2 * _N, _QW), jnp.float32),
        scratch_types=[
            pltpu.VMEM((_BITS_W,), jnp.int32),
            pltpu.VMEM((_NCHUNK, _CW), jnp.int32),
            pltpu.VMEM((_NCHUNK, _CW), jnp.int32),
            pltpu.VMEM((_NBUF, _CW, _QW), jnp.float32),
            pltpu.VMEM_SHARED((_ACC_ROWS, _QW), jnp.float32),
            [pltpu.SemaphoreType.DMA] * _NBUF,
            [pltpu.SemaphoreType.DMA] * _NBUF,
        ],
    )
    def agg(h2_hbm, src_hbm, dst_hbm, bits_hbm, zero_hbm, acc_hbm,
            bits_v, gidx_v, aidx_v, bufs_v, acc_sh, gsems, ssems):
        cid = lax.axis_index("c")
        sid = lax.axis_index("s")
        slab = sid * _SLAB

        pltpu.sync_copy(bits_hbm, bits_v)
        pltpu.sync_copy(src_hbm.at[sid], gidx_v)
        pltpu.sync_copy(dst_hbm.at[sid], aidx_v)

        base = cid * 2 * (2 * _N)

        def idx_body(j, carry):
            for k in range(_CW // 16):
                ds = pl.ds(k * 16, 16)
                s = gidx_v[j, ds]
                d = aidx_v[j, ds]
                dc = jnp.maximum(d, 0)
                sw = plsc.load_gather(bits_v, [lax.shift_right_logical(s, 5)])
                dw = plsc.load_gather(bits_v, [lax.shift_right_logical(dc, 5)])
                s_gene = lax.shift_right_logical(sw, s & 31) & 1
                d_gene = lax.shift_right_logical(dw, dc & 31) & 1
                t = (s_gene == 1) & (d_gene == 0)
                gidx_v[j, ds] = base + s + jnp.where(t, _N, 0)
                aidx_v[j, ds] = jnp.where(d < 0, 2 * _N,
                                          d + jnp.where(s_gene == 0, _N, 0))
            return carry

        def bump_body(j, carry):
            for k in range(_CW // 16):
                ds = pl.ds(k * 16, 16)
                gidx_v[j, ds] = gidx_v[j, ds] + 2 * _N
            return carry

        for q in range(2):
            qg = cid * 2 + q
            lax.fori_loop(0, _NCHUNK, idx_body if q == 0 else bump_body, 0)

            pltpu.sync_copy(zero_hbm, acc_sh.at[pl.ds(slab, _SLAB)])

            @pl.when(sid == _NSUB - 1)
            def _():
                pltpu.sync_copy(zero_hbm.at[pl.ds(0, _SLAB_EXTRA)],
                                acc_sh.at[pl.ds(_NSUB * _SLAB, _SLAB_EXTRA)])

            plsc.subcore_barrier()

            ghand = [None] * _NCHUNK
            shand = [None] * _NCHUNK

            def start_gather(j):
                return pltpu.async_copy(h2_hbm.at[gidx_v.at[j]],
                                        bufs_v.at[j % _NBUF],
                                        gsems[j % _NBUF])

            def start_scatter(j):
                return pltpu.async_copy(bufs_v.at[j % _NBUF],
                                        acc_sh.at[aidx_v.at[j]],
                                        ssems[j % _NBUF], add=True)

            for j in range(_NCHUNK):
                if j >= _NBUF:
                    shand[j - _NBUF].wait()
                ghand[j] = start_gather(j)
                if j >= 2:
                    ghand[j - 2].wait()
                    shand[j - 2] = start_scatter(j - 2)
            for j in range(_NCHUNK - 2, _NCHUNK):
                ghand[j].wait()
                shand[j] = start_scatter(j)
            for j in range(_NCHUNK - _NBUF, _NCHUNK):
                shand[j].wait()

            plsc.subcore_barrier()
            pltpu.sync_copy(acc_sh.at[pl.ds(slab, _SLAB)],
                            acc_hbm.at[qg, pl.ds(slab, _SLAB)])

            @pl.when(sid == _NSUB - 1)
            def _():
                pltpu.sync_copy(
                    acc_sh.at[pl.ds(_NSUB * _SLAB, _SLAB_EXTRA)],
                    acc_hbm.at[qg, pl.ds(_NSUB * _SLAB, _SLAB_EXTRA)])

    return agg(h2, srcp, dstp, bits, zeros)


def kernel(features, edge_index, node_ids, W, bias, alpha):
    n, f = features.shape
    h_dim = W.shape[1]
    gene_num = alpha.shape[0] - 2
    src = edge_index[0]
    dst = edge_index[1]

    srcp = jnp.pad(src.reshape(_NSUB, _E // _NSUB),
                   ((0, 0), (0, _ESUB - _E // _NSUB))).reshape(
                       _NSUB, _NCHUNK, _CW)
    dstp = jnp.pad(dst.reshape(_NSUB, _E // _NSUB),
                   ((0, 0), (0, _ESUB - _E // _NSUB)),
                   constant_values=-1).reshape(_NSUB, _NCHUNK, _CW)
    alpha_flat = jnp.pad(alpha[:, 0], (0, 2008 - alpha.shape[0]))
    zeros1d = jnp.zeros((640,), jnp.float32)

    degs, av1 = _degrees_alpha(srcp, dstp, node_ids, alpha_flat, zeros1d)
    out_deg = jnp.clip(degs[0, 0, :n] + degs[1, 0, :n], 1.0, None)
    in_deg = jnp.clip(degs[0, 1, :n] + degs[1, 1, :n], 1.0, None)
    so = (out_deg ** -0.5)[:, None]
    si = (in_deg ** -0.5)[:, None]

    is_gene = node_ids >= 0
    av = av1[:, None]
    c3 = alpha[gene_num, 0]
    c4 = alpha[gene_num + 1, 0]
    wp = jnp.where(is_gene, c3, 1.0)[:, None]
    wq = jnp.where(is_gene, av1, c4)[:, None]

    bm = 1000
    h2 = pl.pallas_call(
        _fw_table_kernel,
        grid=(n // bm,),
        in_specs=[pl.BlockSpec((bm, f), lambda i: (i, 0)),
                  pl.BlockSpec((f, h_dim), lambda i: (0, 0)),
                  pl.BlockSpec((bm, 1), lambda i: (i, 0)),
                  pl.BlockSpec((bm, 1), lambda i: (i, 0))],
        out_specs=pl.BlockSpec((4, 2, bm, _QW), lambda i: (0, 0, i, 0)),
        out_shape=jax.ShapeDtypeStruct((4, 2, n, _QW), jnp.float32),
    )(features, W, so, av)
    h2 = h2.reshape(4 * 2 * n, _QW)

    zeros = jnp.zeros((_SLAB, _QW), jnp.float32)

    gb = jnp.pad(is_gene, (0, _BITS_W * 32 - n)).reshape(_BITS_W, 32)
    bits = (gb.astype(jnp.uint32) << jnp.arange(32, dtype=jnp.uint32)
            ).sum(axis=1, dtype=jnp.uint32).astype(jnp.int32)

    acc = _edge_agg(h2, srcp, dstp, bits, zeros)

    z, zb = pl.pallas_call(
        _combine_kernel,
        grid=(n // bm,),
        in_specs=[pl.BlockSpec((4, bm, _QW), lambda i: (0, i, 0)),
                  pl.BlockSpec((4, bm, _QW), lambda i: (0, i + _N // 1000, 0)),
                  pl.BlockSpec((bm, 1), lambda i: (i, 0)),
                  pl.BlockSpec((bm, 1), lambda i: (i, 0)),
                  pl.BlockSpec((bm, 1), lambda i: (i, 0)),
                  pl.BlockSpec((1, h_dim), lambda i: (0, 0))],
        out_specs=[pl.BlockSpec((bm, h_dim), lambda i: (i, 0)),
                   pl.BlockSpec((bm, h_dim), lambda i: (i, 0))],
        out_shape=[jax.ShapeDtypeStruct((n, h_dim), jnp.float32),
                   jax.ShapeDtypeStruct((n, h_dim), jnp.bfloat16)],
    )(acc, acc, wp, wq, si, bias[None, :])

    bz = 2560
    adj = pl.pallas_call(
        _zzt_kernel,
        grid=(pl.cdiv(n, bz), pl.cdiv(n, bz)),
        in_specs=[pl.BlockSpec((bz, h_dim), lambda i, j: (i, 0)),
                  pl.BlockSpec((bz, h_dim), lambda i, j: (j, 0))],
        out_specs=pl.BlockSpec((bz, bz), lambda i, j: (i, j)),
        out_shape=jax.ShapeDtypeStruct((n, n), jnp.float32),
    )(zb, zb)
    return (adj, z)
